# baseline (speedup 1.0000x reference)
import functools

import jax
import jax.numpy as jnp
from jax import lax
from jax.experimental import pallas as pl
from jax.experimental.pallas import tpu as pltpu
from jax.experimental.pallas import tpu_sc as plsc


def _make_sc_gather(V, DP, B):
    info = plsc.get_sparse_core_info()
    NC, NS = info.num_cores, info.num_subcores
    NW = NC * NS
    b_per_w = B // NW
    mesh = plsc.VectorSubcoreMesh(core_axis_name="c", subcore_axis_name="s")

    @functools.partial(
        pl.kernel,
        mesh=mesh,
        out_type=jax.ShapeDtypeStruct((B, DP), jnp.float32),
        scratch_types=[
            pltpu.VMEM((b_per_w,), jnp.int32),
            pltpu.VMEM((b_per_w, DP), jnp.float32),
            pltpu.SemaphoreType.DMA,
        ],
    )
    def sc_gather(table_hbm, idx_hbm, out_hbm, idx_v, rows_v, sem):
        wid = lax.axis_index("s") * NC + lax.axis_index("c")
        base = wid * b_per_w
        pltpu.sync_copy(idx_hbm.at[pl.ds(base, b_per_w)], idx_v)
        pltpu.async_copy(table_hbm.at[idx_v], rows_v, sem).wait()
        pltpu.sync_copy(rows_v, out_hbm.at[pl.ds(base, b_per_w)])

    return sc_gather


def _padT_body(tt_ref, o_ref):
    d = tt_ref.shape[0]
    eye = (
        lax.broadcasted_iota(jnp.int32, (d, d), 0)
        == lax.broadcasted_iota(jnp.int32, (d, d), 1)
    ).astype(jnp.float32)
    o_ref[:, :d] = lax.dot_general(
        tt_ref[...],
        eye,
        dimension_numbers=(((0,), (1,)), ((), ())),
        preferred_element_type=jnp.float32,
    )


def _pad_transpose(tableT, vblk):
    D, V = tableT.shape
    return pl.pallas_call(
        _padT_body,
        grid=(pl.cdiv(V, vblk),),
        in_specs=[pl.BlockSpec((D, vblk), lambda i: (0, i))],
        out_specs=pl.BlockSpec((vblk, 128), lambda i: (i, 0)),
        out_shape=jax.ShapeDtypeStruct((V, 128), jnp.float32),
    )(tableT)


def _proj_t_body(wt_ref, xt_ref, b_ref, o_ref):
    acc = lax.dot_general(
        wt_ref[...],
        xt_ref[...],
        dimension_numbers=(((0,), (0,)), ((), ())),
        preferred_element_type=jnp.float32,
    )
    o_ref[...] = acc + jnp.transpose(b_ref[...], (1, 0))


def _tc_project_t(xt, WT, brow, vblk):
    D, B = xt.shape
    V = WT.shape[1]
    return pl.pallas_call(
        _proj_t_body,
        grid=(pl.cdiv(V, vblk),),
        in_specs=[
            pl.BlockSpec((D, vblk), lambda i: (0, i)),
            pl.BlockSpec((D, B), lambda i: (0, 0)),
            pl.BlockSpec((1, vblk), lambda i: (0, i)),
        ],
        out_specs=pl.BlockSpec((vblk, B), lambda i: (i, 0)),
        out_shape=jax.ShapeDtypeStruct((V, B), jnp.float32),
    )(WT, xt, brow)


def kernel(inputs, table, W, b):
    V, D = table.shape
    B = inputs.shape[0]
    idx = inputs.astype(jnp.int32)
    table_p = _pad_transpose(table.T, 2048)
    xp = _make_sc_gather(V, 128, B)(table_p, idx)
    logits_t = _tc_project_t(xp[:, :D].T, W.T, b.reshape(1, V), 4096)
    return logits_t.T


# speedup vs baseline: 1.0109x; 1.0109x over previous
import functools

import jax
import jax.numpy as jnp
from jax import lax
from jax.experimental import pallas as pl
from jax.experimental.pallas import tpu as pltpu
from jax.experimental.pallas import tpu_sc as plsc


def _make_sc_gather(V, DP, B):
    info = plsc.get_sparse_core_info()
    NC, NS = info.num_cores, info.num_subcores
    NW = NC * NS
    b_per_w = B // NW
    mesh = plsc.VectorSubcoreMesh(core_axis_name="c", subcore_axis_name="s")

    @functools.partial(
        pl.kernel,
        mesh=mesh,
        out_type=jax.ShapeDtypeStruct((B, DP), jnp.float32),
        scratch_types=[
            pltpu.VMEM((b_per_w,), jnp.int32),
            pltpu.VMEM((b_per_w, DP), jnp.float32),
            pltpu.SemaphoreType.DMA,
        ],
    )
    def sc_gather(table_hbm, idx_hbm, out_hbm, idx_v, rows_v, sem):
        wid = lax.axis_index("s") * NC + lax.axis_index("c")
        base = wid * b_per_w
        pltpu.sync_copy(idx_hbm.at[pl.ds(base, b_per_w)], idx_v)
        pltpu.async_copy(table_hbm.at[idx_v], rows_v, sem).wait()
        pltpu.sync_copy(rows_v, out_hbm.at[pl.ds(base, b_per_w)])

    return sc_gather


def _padT_body(tt_ref, o_ref):
    o_ref[:, : tt_ref.shape[0]] = jnp.transpose(tt_ref[...], (1, 0))


def _pad_transpose(tableT, vblk):
    D, V = tableT.shape
    return pl.pallas_call(
        _padT_body,
        grid=(pl.cdiv(V, vblk),),
        in_specs=[pl.BlockSpec((D, vblk), lambda i: (0, i))],
        out_specs=pl.BlockSpec((vblk, 128), lambda i: (i, 0)),
        out_shape=jax.ShapeDtypeStruct((V, 128), jnp.float32),
    )(tableT)


def _proj_t_body(wt_ref, xt_ref, b_ref, o_ref):
    acc = lax.dot_general(
        wt_ref[...],
        xt_ref[...],
        dimension_numbers=(((0,), (0,)), ((), ())),
        preferred_element_type=jnp.float32,
    )
    o_ref[...] = acc + jnp.transpose(b_ref[...], (1, 0))


def _tc_project_t(xt, WT, brow, vblk):
    D, B = xt.shape
    V = WT.shape[1]
    return pl.pallas_call(
        _proj_t_body,
        grid=(pl.cdiv(V, vblk),),
        in_specs=[
            pl.BlockSpec((D, vblk), lambda i: (0, i)),
            pl.BlockSpec((D, B), lambda i: (0, 0)),
            pl.BlockSpec((1, vblk), lambda i: (0, i)),
        ],
        out_specs=pl.BlockSpec((vblk, B), lambda i: (i, 0)),
        out_shape=jax.ShapeDtypeStruct((V, B), jnp.float32),
    )(WT, xt, brow)


def kernel(inputs, table, W, b):
    V, D = table.shape
    B = inputs.shape[0]
    idx = inputs.astype(jnp.int32)
    table_p = _pad_transpose(table.T, 2048)
    xp = _make_sc_gather(V, 128, B)(table_p, idx)
    logits_t = _tc_project_t(xp[:, :D].T, W.T, b.reshape(1, V), 4096)
    return logits_t.T
